# baseline (speedup 1.0000x reference)
import jax
import jax.numpy as jnp
from jax.experimental import pallas as pl
from jax.experimental.pallas import tpu as pltpu

_N_GRAPHS = 256
_NODES_PER_G = 24
_EDGES_PER_G = 48
_MAX_N = 32
_MAX_E = 64
_D_STATE = 32
_MSG_OUT = 79
_N_PROP = 5
_TEMP = 0.1
_SINK_ITERS = 20
_LAMBDA = 1.0
_PAIRS = _N_GRAPHS // 2
_PN = 2 * _NODES_PER_G
_PE = 2 * _EDGES_PER_G
_PPB = 32
_BN = _PPB * _PN
_BE = _PPB * _PE
_GP = 4
_NSB = _PPB // _GP
_SN = _GP * _PN
_SE = _GP * _PE
_LP = 4
_SLABS = _PAIRS // _LP
_BSL = _PPB // _LP


def _stage_a(nf, ef, flp, tlp,
             Wne, bne, Wee, bee,
             Wm1s, Wm1d, Wm1e, bm1, Wm2, bm2,
             Wu1h, Wu1a, bu1, Wu2, bu2,
             Ws1, bs1, Ws2, bs2,
             Wl1s, Wl1d, Wl1e, bl1, Wl2, bl2,
             cost_o, rmax_o, nd_o, ed_o):
    f32 = jnp.float32
    h = nf[0] @ Wne[...] + bne[...]
    e = ef[0] @ Wee[...] + bee[...]

    iota_sb = jax.lax.broadcasted_iota(jnp.int32, (_SN, _SE), 0)
    flb = flp[0]
    tlb = tlp[0]
    F_Ts = [(iota_sb + sb * _SN == flb[:, sb * _SE:(sb + 1) * _SE]).astype(f32)
            for sb in range(_NSB)]
    T_Ts = [(iota_sb + sb * _SN == tlb[:, sb * _SE:(sb + 1) * _SE]).astype(f32)
            for sb in range(_NSB)]

    def gatherc(M_Ts, x):
        return jnp.concatenate(
            [jax.lax.dot_general(M_Ts[sb], x[sb * _SN:(sb + 1) * _SN],
                                 (((0,), (0,)), ((), ())),
                                 preferred_element_type=f32)
             for sb in range(_NSB)], axis=0)

    def scatterc(M_Ts, m):
        return jnp.concatenate(
            [M_Ts[sb] @ m[sb * _SE:(sb + 1) * _SE] for sb in range(_NSB)],
            axis=0)

    for _ in range(_N_PROP):
        src = gatherc(F_Ts, h)
        dst = gatherc(T_Ts, h)
        z = src @ Wm1s[...] + dst @ Wm1d[...] + e @ Wm1e[...] + bm1[...]
        m = jnp.maximum(z, 0.0) @ Wm2[...] + bm2[...]
        agg = scatterc(T_Ts, m)
        u = h @ Wu1h[...] + agg @ Wu1a[...] + bu1[...]
        h = jnp.maximum(u, 0.0) @ Wu2[...] + bu2[...]

    r_row = jax.lax.broadcasted_iota(jnp.int32, (_PPB * _MAX_N, _BN), 0)
    r_col = jax.lax.broadcasted_iota(jnp.int32, (_PPB * _MAX_N, _BN), 1)
    b_id = r_row // _MAX_N
    i_id = r_row % _MAX_N
    valid = i_id < _NODES_PER_G
    qS = ((r_col == b_id * _PN + i_id) & valid).astype(f32)
    cS = ((r_col == b_id * _PN + _NODES_PER_G + i_id) & valid).astype(f32)
    qn_all = qS @ h
    cn_all = cS @ h

    tq_all = jnp.maximum(qn_all @ Ws1[...] + bs1[...], 0.0) @ Ws2[...] + bs2[...]
    tc_all = jnp.maximum(cn_all @ Ws1[...] + bs1[...], 0.0) @ Ws2[...] + bs2[...]

    tq3 = tq_all.reshape(_PPB, _MAX_N, _MAX_N)
    tc3 = tc_all.reshape(_PPB, _MAX_N, _MAX_N)
    cost4 = jnp.sum(jnp.abs(tq3[:, :, None, :] - tc3[:, None, :, :]), axis=-1)

    rmin4 = jnp.min(cost4, axis=-1, keepdims=True)
    slabs_c = []
    slabs_m = []
    for s in range(_BSL):
        slabs_c.append(jnp.concatenate(
            [cost4[s * _LP + p] for p in range(_LP)], axis=1))
        slabs_m.append(jnp.concatenate(
            [jnp.broadcast_to(rmin4[s * _LP + p], (_MAX_N, _MAX_N))
             for p in range(_LP)], axis=1))
    cost_o[...] = jnp.concatenate(slabs_c, axis=0)
    rmax_o[...] = jnp.concatenate(slabs_m, axis=0)

    qn3 = qn_all.reshape(_PPB, _MAX_N, _MAX_N)
    cn3 = cn_all.reshape(_PPB, _MAX_N, _MAX_N)
    nd_o[...] = jnp.sum(jnp.abs(qn3[:, :, None, :] - cn3[:, None, :, :]),
                        axis=-1)

    src = gatherc(F_Ts, h)
    dst = gatherc(T_Ts, h)
    z1 = src @ Wl1s[...] + dst @ Wl1d[...] + e @ Wl1e[...] + bl1[...]
    z2 = dst @ Wl1s[...] + src @ Wl1d[...] + e @ Wl1e[...] + bl1[...]
    em = (jnp.maximum(z1, 0.0) + jnp.maximum(z2, 0.0)) @ Wl2[...] + 2.0 * bl2[...]

    pade = jnp.zeros((_MAX_E - _EDGES_PER_G, _MSG_OUT), jnp.float32)
    for b in range(_PPB):
        e0 = b * _PE
        qe = jnp.concatenate([em[e0:e0 + _EDGES_PER_G], pade], axis=0)
        ce = jnp.concatenate([em[e0 + _EDGES_PER_G:e0 + _PE], pade], axis=0)
        ed_o[b] = jnp.sum(jnp.abs(qe[:, None, :] - ce[None, :, :]), axis=-1)


def _stage_b(cost, rmax, P_o):
    f32 = jnp.float32
    la = -cost[...] / _TEMP
    m0 = -rmax[...] / _TEMP

    gi = jax.lax.broadcasted_iota(jnp.int32, (128, 128), 0)
    gj = jax.lax.broadcasted_iota(jnp.int32, (128, 128), 1)
    G = (gi // _MAX_N == gj // _MAX_N).astype(f32)

    def group_sums(x):
        return jax.lax.dot_general(x, G, (((1,), (0,)), ((), ())),
                                   preferred_element_type=f32)

    def col_lse_sub(x2):
        x3 = x2.reshape(_SLABS, _MAX_N, 128)
        m = jnp.max(x3, axis=1, keepdims=True)
        s = jnp.sum(jnp.exp(x3 - m), axis=1, keepdims=True)
        return (x3 - (m + jnp.log(s))).reshape(_SLABS * _MAX_N, 128)

    la = la - (m0 + jnp.log(group_sums(jnp.exp(la - m0))))
    la = col_lse_sub(la)
    for _ in range(_SINK_ITERS - 1):
        la = la - jnp.log(group_sums(jnp.exp(la)))
        la = col_lse_sub(la)
    P_o[...] = jnp.exp(la)


def _stage_c(Pp, nd, ed, qf, qt, cf, ct, out):
    f32 = jnp.float32
    iota_k = jax.lax.broadcasted_iota(jnp.int32, (_MAX_N, _MAX_E), 0)
    qfb, qtb, cfb, ctb = qf[0], qt[0], cf[0], ct[0]

    def gather(M_T, x):
        return jax.lax.dot_general(M_T, x, (((0,), (0,)), ((), ())),
                                   preferred_element_type=f32)

    Pblk = Pp[...]
    for b in range(_PPB):
        s, p4 = b // _LP, b % _LP
        P = Pblk[s * _MAX_N:(s + 1) * _MAX_N,
                 p4 * _MAX_N:(p4 + 1) * _MAX_N]
        A_T = (iota_k == qfb[b:b + 1]).astype(f32)
        B_T = (iota_k == qtb[b:b + 1]).astype(f32)
        C_T = (iota_k == cfb[b:b + 1]).astype(f32)
        D_T = (iota_k == ctb[b:b + 1]).astype(f32)
        rowsA = gather(A_T, P)
        rowsB = gather(B_T, P)
        plan = jnp.maximum((rowsA @ C_T) * (rowsB @ D_T),
                           (rowsA @ D_T) * (rowsB @ C_T))
        val = jnp.sum(plan * ed[b]) + _LAMBDA * jnp.sum(P * nd[b])
        out[b] = jnp.full((8, 128), val, f32)


def kernel(node_features, edge_features, from_idx, to_idx, graph_idx,
           graph_sizes, W_ne, b_ne, W_ee, b_ee, W_m1, b_m1, W_m2, b_m2,
           W_u1, b_u1, W_u2, b_u2, W_s1, b_s1, W_s2, b_s2,
           W_l1, b_l1, W_l2, b_l2):
    f32 = jnp.float32
    nblocks = _PAIRS // _PPB
    nf3 = node_features.reshape(nblocks, _BN, -1)
    ef3 = edge_features.reshape(nblocks, _BE, -1)

    blk_offs = (jnp.arange(nblocks, dtype=jnp.int32) * _BN)[:, None]
    flp = (from_idx.reshape(nblocks, _BE) - blk_offs).reshape(nblocks, 1, _BE)
    tlp = (to_idx.reshape(nblocks, _BE) - blk_offs).reshape(nblocks, 1, _BE)

    g_offs = (jnp.arange(_N_GRAPHS, dtype=jnp.int32) * _NODES_PER_G)[:, None]
    fg = from_idx.reshape(_N_GRAPHS, _EDGES_PER_G) - g_offs
    tg = to_idx.reshape(_N_GRAPHS, _EDGES_PER_G) - g_offs
    pad = ((0, 0), (0, _MAX_E - _EDGES_PER_G))
    fg = jnp.pad(fg, pad, constant_values=_NODES_PER_G)
    tg = jnp.pad(tg, pad, constant_values=_NODES_PER_G)
    qf = fg[0::2].reshape(nblocks, _PPB, _MAX_E)
    qt = tg[0::2].reshape(nblocks, _PPB, _MAX_E)
    cf = fg[1::2].reshape(nblocks, _PPB, _MAX_E)
    ct = tg[1::2].reshape(nblocks, _PPB, _MAX_E)

    Wm1s, Wm1d, Wm1e = W_m1[:32], W_m1[32:64], W_m1[64:]
    Wu1h, Wu1a = W_u1[:32], W_u1[32:]
    Wl1s, Wl1d, Wl1e = W_l1[:32], W_l1[32:64], W_l1[64:]

    def row(b):
        return b.reshape(1, -1)

    a_inputs = [nf3, ef3, flp, tlp,
                W_ne, row(b_ne), W_ee, row(b_ee),
                Wm1s, Wm1d, Wm1e, row(b_m1), W_m2, row(b_m2),
                Wu1h, Wu1a, row(b_u1), W_u2, row(b_u2),
                W_s1, row(b_s1), W_s2, row(b_s2),
                Wl1s, Wl1d, Wl1e, row(b_l1), W_l2, row(b_l2)]

    def bspec(x):
        if x.ndim == 3:
            return pl.BlockSpec((1,) + x.shape[1:], lambda p: (p, 0, 0))
        return pl.BlockSpec(x.shape, lambda p: (0,) * x.ndim)

    packed_rows = _BSL * _MAX_N

    cost_p, rmax_p, nd3, ed3 = pl.pallas_call(
        _stage_a,
        grid=(nblocks,),
        in_specs=[bspec(x) for x in a_inputs],
        out_specs=[
            pl.BlockSpec((packed_rows, 128), lambda p: (p, 0)),
            pl.BlockSpec((packed_rows, 128), lambda p: (p, 0)),
            pl.BlockSpec((_PPB, _MAX_N, _MAX_N), lambda p: (p, 0, 0)),
            pl.BlockSpec((_PPB, _MAX_E, _MAX_E), lambda p: (p, 0, 0)),
        ],
        out_shape=[
            jax.ShapeDtypeStruct((_SLABS * _MAX_N, 128), f32),
            jax.ShapeDtypeStruct((_SLABS * _MAX_N, 128), f32),
            jax.ShapeDtypeStruct((_PAIRS, _MAX_N, _MAX_N), f32),
            jax.ShapeDtypeStruct((_PAIRS, _MAX_E, _MAX_E), f32),
        ],
    )(*a_inputs)

    Pp = pl.pallas_call(
        _stage_b,
        grid=(1,),
        in_specs=[pl.BlockSpec((_SLABS * _MAX_N, 128), lambda p: (0, 0)),
                  pl.BlockSpec((_SLABS * _MAX_N, 128), lambda p: (0, 0))],
        out_specs=pl.BlockSpec((_SLABS * _MAX_N, 128), lambda p: (0, 0)),
        out_shape=jax.ShapeDtypeStruct((_SLABS * _MAX_N, 128), f32),
    )(cost_p, rmax_p)

    c_inputs = [Pp, nd3, ed3, qf, qt, cf, ct]
    c_specs = [
        pl.BlockSpec((packed_rows, 128), lambda p: (p, 0)),
        pl.BlockSpec((_PPB, _MAX_N, _MAX_N), lambda p: (p, 0, 0)),
        pl.BlockSpec((_PPB, _MAX_E, _MAX_E), lambda p: (p, 0, 0)),
        pl.BlockSpec((1, _PPB, _MAX_E), lambda p: (p, 0, 0)),
        pl.BlockSpec((1, _PPB, _MAX_E), lambda p: (p, 0, 0)),
        pl.BlockSpec((1, _PPB, _MAX_E), lambda p: (p, 0, 0)),
        pl.BlockSpec((1, _PPB, _MAX_E), lambda p: (p, 0, 0)),
    ]
    out3 = pl.pallas_call(
        _stage_c,
        grid=(nblocks,),
        in_specs=c_specs,
        out_specs=pl.BlockSpec((_PPB, 8, 128), lambda p: (p, 0, 0)),
        out_shape=jax.ShapeDtypeStruct((_PAIRS, 8, 128), f32),
    )(*c_inputs)
    return out3[:, 0, 0]


# speedup vs baseline: 1.3039x; 1.3039x over previous
import jax
import jax.numpy as jnp
from jax.experimental import pallas as pl
from jax.experimental.pallas import tpu as pltpu

_N_GRAPHS = 256
_NODES_PER_G = 24
_EDGES_PER_G = 48
_MAX_N = 32
_MAX_E = 64
_D_STATE = 32
_MSG_OUT = 79
_N_PROP = 5
_TEMP = 0.1
_SINK_ITERS = 20
_LAMBDA = 1.0
_PAIRS = _N_GRAPHS // 2
_PN = 2 * _NODES_PER_G
_PE = 2 * _EDGES_PER_G
_PPB = 16
_BN = _PPB * _PN
_BE = _PPB * _PE
_GP = 2
_NSB = _PPB // _GP
_SN = _GP * _PN
_SE = _GP * _PE
_LP = 4
_SLABS = _PAIRS // _LP
_BSL = _PPB // _LP


def _stage_a(nf, ef, flp, tlp,
             Wne, bne, Wee, bee,
             Wm1s, Wm1d, Wm1e, bm1, Wm2, bm2,
             Wu1h, Wu1a, bu1, Wu2, bu2,
             Ws1, bs1, Ws2, bs2,
             Wl1s, Wl1d, Wl1e, bl1, Wl2, bl2,
             cost_o, rmax_o, nd_o, ed_o):
    f32 = jnp.float32
    h = nf[0] @ Wne[...] + bne[...]
    e = ef[0] @ Wee[...] + bee[...]

    iota_sb = jax.lax.broadcasted_iota(jnp.int32, (_SN, _SE), 0)
    flb = flp[0]
    tlb = tlp[0]
    F_Ts = [(iota_sb + sb * _SN == flb[:, sb * _SE:(sb + 1) * _SE]).astype(f32)
            for sb in range(_NSB)]
    T_Ts = [(iota_sb + sb * _SN == tlb[:, sb * _SE:(sb + 1) * _SE]).astype(f32)
            for sb in range(_NSB)]

    def gatherc(M_Ts, x):
        return jnp.concatenate(
            [jax.lax.dot_general(M_Ts[sb], x[sb * _SN:(sb + 1) * _SN],
                                 (((0,), (0,)), ((), ())),
                                 preferred_element_type=f32)
             for sb in range(_NSB)], axis=0)

    def scatterc(M_Ts, m):
        return jnp.concatenate(
            [M_Ts[sb] @ m[sb * _SE:(sb + 1) * _SE] for sb in range(_NSB)],
            axis=0)

    for _ in range(_N_PROP):
        src = gatherc(F_Ts, h)
        dst = gatherc(T_Ts, h)
        z = src @ Wm1s[...] + dst @ Wm1d[...] + e @ Wm1e[...] + bm1[...]
        m = jnp.maximum(z, 0.0) @ Wm2[...] + bm2[...]
        agg = scatterc(T_Ts, m)
        u = h @ Wu1h[...] + agg @ Wu1a[...] + bu1[...]
        h = jnp.maximum(u, 0.0) @ Wu2[...] + bu2[...]

    r_row = jax.lax.broadcasted_iota(jnp.int32, (_PPB * _MAX_N, _BN), 0)
    r_col = jax.lax.broadcasted_iota(jnp.int32, (_PPB * _MAX_N, _BN), 1)
    b_id = r_row // _MAX_N
    i_id = r_row % _MAX_N
    valid = i_id < _NODES_PER_G
    qS = ((r_col == b_id * _PN + i_id) & valid).astype(f32)
    cS = ((r_col == b_id * _PN + _NODES_PER_G + i_id) & valid).astype(f32)
    qn_all = qS @ h
    cn_all = cS @ h

    tq_all = jnp.maximum(qn_all @ Ws1[...] + bs1[...], 0.0) @ Ws2[...] + bs2[...]
    tc_all = jnp.maximum(cn_all @ Ws1[...] + bs1[...], 0.0) @ Ws2[...] + bs2[...]

    tq3 = tq_all.reshape(_PPB, _MAX_N, _MAX_N)
    tc3 = tc_all.reshape(_PPB, _MAX_N, _MAX_N)
    cost4 = jnp.sum(jnp.abs(tq3[:, :, None, :] - tc3[:, None, :, :]), axis=-1)

    rmin4 = jnp.min(cost4, axis=-1, keepdims=True)
    slabs_c = []
    slabs_m = []
    for s in range(_BSL):
        slabs_c.append(jnp.concatenate(
            [cost4[s * _LP + p] for p in range(_LP)], axis=1))
        slabs_m.append(jnp.concatenate(
            [jnp.broadcast_to(rmin4[s * _LP + p], (_MAX_N, _MAX_N))
             for p in range(_LP)], axis=1))
    cost_o[...] = jnp.concatenate(slabs_c, axis=0)
    rmax_o[...] = jnp.concatenate(slabs_m, axis=0)

    qn3 = qn_all.reshape(_PPB, _MAX_N, _MAX_N)
    cn3 = cn_all.reshape(_PPB, _MAX_N, _MAX_N)
    nd_o[...] = jnp.sum(jnp.abs(qn3[:, :, None, :] - cn3[:, None, :, :]),
                        axis=-1)

    src = gatherc(F_Ts, h)
    dst = gatherc(T_Ts, h)
    z1 = src @ Wl1s[...] + dst @ Wl1d[...] + e @ Wl1e[...] + bl1[...]
    z2 = dst @ Wl1s[...] + src @ Wl1d[...] + e @ Wl1e[...] + bl1[...]
    em = (jnp.maximum(z1, 0.0) + jnp.maximum(z2, 0.0)) @ Wl2[...] + 2.0 * bl2[...]

    pade = jnp.zeros((_MAX_E - _EDGES_PER_G, _MSG_OUT), jnp.float32)
    for b in range(_PPB):
        e0 = b * _PE
        qe = jnp.concatenate([em[e0:e0 + _EDGES_PER_G], pade], axis=0)
        ce = jnp.concatenate([em[e0 + _EDGES_PER_G:e0 + _PE], pade], axis=0)
        ed_o[b] = jnp.sum(jnp.abs(qe[:, None, :] - ce[None, :, :]), axis=-1)


def _stage_b(cost, rmax, P_o):
    f32 = jnp.float32
    la = -cost[...] / _TEMP
    m0 = -rmax[...] / _TEMP

    gi = jax.lax.broadcasted_iota(jnp.int32, (128, 128), 0)
    gj = jax.lax.broadcasted_iota(jnp.int32, (128, 128), 1)
    G = (gi // _MAX_N == gj // _MAX_N).astype(f32)

    def group_sums(x):
        return jax.lax.dot_general(x, G, (((1,), (0,)), ((), ())),
                                   preferred_element_type=f32)

    def col_lse_sub(x2):
        x3 = x2.reshape(_SLABS, _MAX_N, 128)
        m = jnp.max(x3, axis=1, keepdims=True)
        s = jnp.sum(jnp.exp(x3 - m), axis=1, keepdims=True)
        return (x3 - (m + jnp.log(s))).reshape(_SLABS * _MAX_N, 128)

    la = la - (m0 + jnp.log(group_sums(jnp.exp(la - m0))))
    la = col_lse_sub(la)
    for _ in range(_SINK_ITERS - 1):
        la = la - jnp.log(group_sums(jnp.exp(la)))
        la = col_lse_sub(la)
    P_o[...] = jnp.exp(la)


def _stage_c(Pp, nd, ed, qf, qt, cf, ct, out):
    f32 = jnp.float32
    iota_k = jax.lax.broadcasted_iota(jnp.int32, (_MAX_N, _MAX_E), 0)
    qfb, qtb, cfb, ctb = qf[0], qt[0], cf[0], ct[0]

    def gather(M_T, x):
        return jax.lax.dot_general(M_T, x, (((0,), (0,)), ((), ())),
                                   preferred_element_type=f32)

    Pblk = Pp[...]
    for b in range(_PPB):
        s, p4 = b // _LP, b % _LP
        P = Pblk[s * _MAX_N:(s + 1) * _MAX_N,
                 p4 * _MAX_N:(p4 + 1) * _MAX_N]
        A_T = (iota_k == qfb[b:b + 1]).astype(f32)
        B_T = (iota_k == qtb[b:b + 1]).astype(f32)
        C_T = (iota_k == cfb[b:b + 1]).astype(f32)
        D_T = (iota_k == ctb[b:b + 1]).astype(f32)
        rowsA = gather(A_T, P)
        rowsB = gather(B_T, P)
        plan = jnp.maximum((rowsA @ C_T) * (rowsB @ D_T),
                           (rowsA @ D_T) * (rowsB @ C_T))
        val = jnp.sum(plan * ed[b]) + _LAMBDA * jnp.sum(P * nd[b])
        out[b] = jnp.full((8, 128), val, f32)


def kernel(node_features, edge_features, from_idx, to_idx, graph_idx,
           graph_sizes, W_ne, b_ne, W_ee, b_ee, W_m1, b_m1, W_m2, b_m2,
           W_u1, b_u1, W_u2, b_u2, W_s1, b_s1, W_s2, b_s2,
           W_l1, b_l1, W_l2, b_l2):
    f32 = jnp.float32
    nblocks = _PAIRS // _PPB
    nf3 = node_features.reshape(nblocks, _BN, -1)
    ef3 = edge_features.reshape(nblocks, _BE, -1)

    blk_offs = (jnp.arange(nblocks, dtype=jnp.int32) * _BN)[:, None]
    flp = (from_idx.reshape(nblocks, _BE) - blk_offs).reshape(nblocks, 1, _BE)
    tlp = (to_idx.reshape(nblocks, _BE) - blk_offs).reshape(nblocks, 1, _BE)

    g_offs = (jnp.arange(_N_GRAPHS, dtype=jnp.int32) * _NODES_PER_G)[:, None]
    fg = from_idx.reshape(_N_GRAPHS, _EDGES_PER_G) - g_offs
    tg = to_idx.reshape(_N_GRAPHS, _EDGES_PER_G) - g_offs
    pad = ((0, 0), (0, _MAX_E - _EDGES_PER_G))
    fg = jnp.pad(fg, pad, constant_values=_NODES_PER_G)
    tg = jnp.pad(tg, pad, constant_values=_NODES_PER_G)
    qf = fg[0::2].reshape(nblocks, _PPB, _MAX_E)
    qt = tg[0::2].reshape(nblocks, _PPB, _MAX_E)
    cf = fg[1::2].reshape(nblocks, _PPB, _MAX_E)
    ct = tg[1::2].reshape(nblocks, _PPB, _MAX_E)

    Wm1s, Wm1d, Wm1e = W_m1[:32], W_m1[32:64], W_m1[64:]
    Wu1h, Wu1a = W_u1[:32], W_u1[32:]
    Wl1s, Wl1d, Wl1e = W_l1[:32], W_l1[32:64], W_l1[64:]

    def row(b):
        return b.reshape(1, -1)

    a_inputs = [nf3, ef3, flp, tlp,
                W_ne, row(b_ne), W_ee, row(b_ee),
                Wm1s, Wm1d, Wm1e, row(b_m1), W_m2, row(b_m2),
                Wu1h, Wu1a, row(b_u1), W_u2, row(b_u2),
                W_s1, row(b_s1), W_s2, row(b_s2),
                Wl1s, Wl1d, Wl1e, row(b_l1), W_l2, row(b_l2)]

    def bspec(x):
        if x.ndim == 3:
            return pl.BlockSpec((1,) + x.shape[1:], lambda p: (p, 0, 0))
        return pl.BlockSpec(x.shape, lambda p: (0,) * x.ndim)

    packed_rows = _BSL * _MAX_N

    cost_p, rmax_p, nd3, ed3 = pl.pallas_call(
        _stage_a,
        grid=(nblocks,),
        in_specs=[bspec(x) for x in a_inputs],
        out_specs=[
            pl.BlockSpec((packed_rows, 128), lambda p: (p, 0)),
            pl.BlockSpec((packed_rows, 128), lambda p: (p, 0)),
            pl.BlockSpec((_PPB, _MAX_N, _MAX_N), lambda p: (p, 0, 0)),
            pl.BlockSpec((_PPB, _MAX_E, _MAX_E), lambda p: (p, 0, 0)),
        ],
        out_shape=[
            jax.ShapeDtypeStruct((_SLABS * _MAX_N, 128), f32),
            jax.ShapeDtypeStruct((_SLABS * _MAX_N, 128), f32),
            jax.ShapeDtypeStruct((_PAIRS, _MAX_N, _MAX_N), f32),
            jax.ShapeDtypeStruct((_PAIRS, _MAX_E, _MAX_E), f32),
        ],
    )(*a_inputs)

    Pp = pl.pallas_call(
        _stage_b,
        grid=(1,),
        in_specs=[pl.BlockSpec((_SLABS * _MAX_N, 128), lambda p: (0, 0)),
                  pl.BlockSpec((_SLABS * _MAX_N, 128), lambda p: (0, 0))],
        out_specs=pl.BlockSpec((_SLABS * _MAX_N, 128), lambda p: (0, 0)),
        out_shape=jax.ShapeDtypeStruct((_SLABS * _MAX_N, 128), f32),
    )(cost_p, rmax_p)

    c_inputs = [Pp, nd3, ed3, qf, qt, cf, ct]
    c_specs = [
        pl.BlockSpec((packed_rows, 128), lambda p: (p, 0)),
        pl.BlockSpec((_PPB, _MAX_N, _MAX_N), lambda p: (p, 0, 0)),
        pl.BlockSpec((_PPB, _MAX_E, _MAX_E), lambda p: (p, 0, 0)),
        pl.BlockSpec((1, _PPB, _MAX_E), lambda p: (p, 0, 0)),
        pl.BlockSpec((1, _PPB, _MAX_E), lambda p: (p, 0, 0)),
        pl.BlockSpec((1, _PPB, _MAX_E), lambda p: (p, 0, 0)),
        pl.BlockSpec((1, _PPB, _MAX_E), lambda p: (p, 0, 0)),
    ]
    out3 = pl.pallas_call(
        _stage_c,
        grid=(nblocks,),
        in_specs=c_specs,
        out_specs=pl.BlockSpec((_PPB, 8, 128), lambda p: (p, 0, 0)),
        out_shape=jax.ShapeDtypeStruct((_PAIRS, 8, 128), f32),
    )(*c_inputs)
    return out3[:, 0, 0]


# speedup vs baseline: 1.3118x; 1.0061x over previous
import jax
import jax.numpy as jnp
from jax.experimental import pallas as pl
from jax.experimental.pallas import tpu as pltpu

_N_GRAPHS = 256
_NODES_PER_G = 24
_EDGES_PER_G = 48
_MAX_N = 32
_MAX_E = 64
_D_STATE = 32
_MSG_OUT = 79
_N_PROP = 5
_TEMP = 0.1
_SINK_ITERS = 20
_LAMBDA = 1.0
_PAIRS = _N_GRAPHS // 2
_PN = 2 * _NODES_PER_G
_PE = 2 * _EDGES_PER_G
_PPB = 16
_BN = _PPB * _PN
_BE = _PPB * _PE
_GP = 1
_NSB = _PPB // _GP
_SN = _GP * _PN
_SE = _GP * _PE
_LP = 4
_SLABS = _PAIRS // _LP
_BSL = _PPB // _LP


def _stage_a(nf, ef, flp, tlp,
             Wne, bne, Wee, bee,
             Wm1s, Wm1d, Wm1e, bm1, Wm2, bm2,
             Wu1h, Wu1a, bu1, Wu2, bu2,
             Ws1, bs1, Ws2, bs2,
             Wl1s, Wl1d, Wl1e, bl1, Wl2, bl2,
             cost_o, rmax_o, nd_o, ed_o):
    f32 = jnp.float32
    h = nf[0] @ Wne[...] + bne[...]
    e = ef[0] @ Wee[...] + bee[...]

    iota_sb = jax.lax.broadcasted_iota(jnp.int32, (_SN, _SE), 0)
    flb = flp[0]
    tlb = tlp[0]
    F_Ts = [(iota_sb + sb * _SN == flb[:, sb * _SE:(sb + 1) * _SE]).astype(f32)
            for sb in range(_NSB)]
    T_Ts = [(iota_sb + sb * _SN == tlb[:, sb * _SE:(sb + 1) * _SE]).astype(f32)
            for sb in range(_NSB)]

    def gatherc(M_Ts, x):
        return jnp.concatenate(
            [jax.lax.dot_general(M_Ts[sb], x[sb * _SN:(sb + 1) * _SN],
                                 (((0,), (0,)), ((), ())),
                                 preferred_element_type=f32)
             for sb in range(_NSB)], axis=0)

    def scatterc(M_Ts, m):
        return jnp.concatenate(
            [M_Ts[sb] @ m[sb * _SE:(sb + 1) * _SE] for sb in range(_NSB)],
            axis=0)

    for _ in range(_N_PROP):
        src = gatherc(F_Ts, h)
        dst = gatherc(T_Ts, h)
        z = src @ Wm1s[...] + dst @ Wm1d[...] + e @ Wm1e[...] + bm1[...]
        m = jnp.maximum(z, 0.0) @ Wm2[...] + bm2[...]
        agg = scatterc(T_Ts, m)
        u = h @ Wu1h[...] + agg @ Wu1a[...] + bu1[...]
        h = jnp.maximum(u, 0.0) @ Wu2[...] + bu2[...]

    r_row = jax.lax.broadcasted_iota(jnp.int32, (_PPB * _MAX_N, _BN), 0)
    r_col = jax.lax.broadcasted_iota(jnp.int32, (_PPB * _MAX_N, _BN), 1)
    b_id = r_row // _MAX_N
    i_id = r_row % _MAX_N
    valid = i_id < _NODES_PER_G
    qS = ((r_col == b_id * _PN + i_id) & valid).astype(f32)
    cS = ((r_col == b_id * _PN + _NODES_PER_G + i_id) & valid).astype(f32)
    qn_all = qS @ h
    cn_all = cS @ h

    tq_all = jnp.maximum(qn_all @ Ws1[...] + bs1[...], 0.0) @ Ws2[...] + bs2[...]
    tc_all = jnp.maximum(cn_all @ Ws1[...] + bs1[...], 0.0) @ Ws2[...] + bs2[...]

    tq3 = tq_all.reshape(_PPB, _MAX_N, _MAX_N)
    tc3 = tc_all.reshape(_PPB, _MAX_N, _MAX_N)
    cost4 = jnp.sum(jnp.abs(tq3[:, :, None, :] - tc3[:, None, :, :]), axis=-1)

    rmin4 = jnp.min(cost4, axis=-1, keepdims=True)
    slabs_c = []
    slabs_m = []
    for s in range(_BSL):
        slabs_c.append(jnp.concatenate(
            [cost4[s * _LP + p] for p in range(_LP)], axis=1))
        slabs_m.append(jnp.concatenate(
            [jnp.broadcast_to(rmin4[s * _LP + p], (_MAX_N, _MAX_N))
             for p in range(_LP)], axis=1))
    cost_o[...] = jnp.concatenate(slabs_c, axis=0)
    rmax_o[...] = jnp.concatenate(slabs_m, axis=0)

    qn3 = qn_all.reshape(_PPB, _MAX_N, _MAX_N)
    cn3 = cn_all.reshape(_PPB, _MAX_N, _MAX_N)
    nd_o[...] = jnp.sum(jnp.abs(qn3[:, :, None, :] - cn3[:, None, :, :]),
                        axis=-1)

    src = gatherc(F_Ts, h)
    dst = gatherc(T_Ts, h)
    z1 = src @ Wl1s[...] + dst @ Wl1d[...] + e @ Wl1e[...] + bl1[...]
    z2 = dst @ Wl1s[...] + src @ Wl1d[...] + e @ Wl1e[...] + bl1[...]
    em = (jnp.maximum(z1, 0.0) + jnp.maximum(z2, 0.0)) @ Wl2[...] + 2.0 * bl2[...]

    pade = jnp.zeros((_MAX_E - _EDGES_PER_G, _MSG_OUT), jnp.float32)
    for b in range(_PPB):
        e0 = b * _PE
        qe = jnp.concatenate([em[e0:e0 + _EDGES_PER_G], pade], axis=0)
        ce = jnp.concatenate([em[e0 + _EDGES_PER_G:e0 + _PE], pade], axis=0)
        ed_o[b] = jnp.sum(jnp.abs(qe[:, None, :] - ce[None, :, :]), axis=-1)


def _stage_b(cost, rmax, P_o):
    f32 = jnp.float32
    la = -cost[...] / _TEMP
    m0 = -rmax[...] / _TEMP

    gi = jax.lax.broadcasted_iota(jnp.int32, (128, 128), 0)
    gj = jax.lax.broadcasted_iota(jnp.int32, (128, 128), 1)
    G = (gi // _MAX_N == gj // _MAX_N).astype(f32)

    def group_sums(x):
        return jax.lax.dot_general(x, G, (((1,), (0,)), ((), ())),
                                   preferred_element_type=f32)

    def col_lse_sub(x2):
        x3 = x2.reshape(_SLABS, _MAX_N, 128)
        m = jnp.max(x3, axis=1, keepdims=True)
        s = jnp.sum(jnp.exp(x3 - m), axis=1, keepdims=True)
        return (x3 - (m + jnp.log(s))).reshape(_SLABS * _MAX_N, 128)

    la = la - (m0 + jnp.log(group_sums(jnp.exp(la - m0))))
    la = col_lse_sub(la)
    for _ in range(_SINK_ITERS - 1):
        la = la - jnp.log(group_sums(jnp.exp(la)))
        la = col_lse_sub(la)
    P_o[...] = jnp.exp(la)


def _stage_c(Pp, nd, ed, qf, qt, cf, ct, out):
    f32 = jnp.float32
    iota_k = jax.lax.broadcasted_iota(jnp.int32, (_MAX_N, _MAX_E), 0)
    qfb, qtb, cfb, ctb = qf[0], qt[0], cf[0], ct[0]

    def gather(M_T, x):
        return jax.lax.dot_general(M_T, x, (((0,), (0,)), ((), ())),
                                   preferred_element_type=f32)

    Pblk = Pp[...]
    for b in range(_PPB):
        s, p4 = b // _LP, b % _LP
        P = Pblk[s * _MAX_N:(s + 1) * _MAX_N,
                 p4 * _MAX_N:(p4 + 1) * _MAX_N]
        A_T = (iota_k == qfb[b:b + 1]).astype(f32)
        B_T = (iota_k == qtb[b:b + 1]).astype(f32)
        C_T = (iota_k == cfb[b:b + 1]).astype(f32)
        D_T = (iota_k == ctb[b:b + 1]).astype(f32)
        rowsA = gather(A_T, P)
        rowsB = gather(B_T, P)
        plan = jnp.maximum((rowsA @ C_T) * (rowsB @ D_T),
                           (rowsA @ D_T) * (rowsB @ C_T))
        val = jnp.sum(plan * ed[b]) + _LAMBDA * jnp.sum(P * nd[b])
        out[b] = jnp.full((8, 128), val, f32)


def kernel(node_features, edge_features, from_idx, to_idx, graph_idx,
           graph_sizes, W_ne, b_ne, W_ee, b_ee, W_m1, b_m1, W_m2, b_m2,
           W_u1, b_u1, W_u2, b_u2, W_s1, b_s1, W_s2, b_s2,
           W_l1, b_l1, W_l2, b_l2):
    f32 = jnp.float32
    nblocks = _PAIRS // _PPB
    nf3 = node_features.reshape(nblocks, _BN, -1)
    ef3 = edge_features.reshape(nblocks, _BE, -1)

    blk_offs = (jnp.arange(nblocks, dtype=jnp.int32) * _BN)[:, None]
    flp = (from_idx.reshape(nblocks, _BE) - blk_offs).reshape(nblocks, 1, _BE)
    tlp = (to_idx.reshape(nblocks, _BE) - blk_offs).reshape(nblocks, 1, _BE)

    g_offs = (jnp.arange(_N_GRAPHS, dtype=jnp.int32) * _NODES_PER_G)[:, None]
    fg = from_idx.reshape(_N_GRAPHS, _EDGES_PER_G) - g_offs
    tg = to_idx.reshape(_N_GRAPHS, _EDGES_PER_G) - g_offs
    pad = ((0, 0), (0, _MAX_E - _EDGES_PER_G))
    fg = jnp.pad(fg, pad, constant_values=_NODES_PER_G)
    tg = jnp.pad(tg, pad, constant_values=_NODES_PER_G)
    qf = fg[0::2].reshape(nblocks, _PPB, _MAX_E)
    qt = tg[0::2].reshape(nblocks, _PPB, _MAX_E)
    cf = fg[1::2].reshape(nblocks, _PPB, _MAX_E)
    ct = tg[1::2].reshape(nblocks, _PPB, _MAX_E)

    Wm1s, Wm1d, Wm1e = W_m1[:32], W_m1[32:64], W_m1[64:]
    Wu1h, Wu1a = W_u1[:32], W_u1[32:]
    Wl1s, Wl1d, Wl1e = W_l1[:32], W_l1[32:64], W_l1[64:]

    def row(b):
        return b.reshape(1, -1)

    a_inputs = [nf3, ef3, flp, tlp,
                W_ne, row(b_ne), W_ee, row(b_ee),
                Wm1s, Wm1d, Wm1e, row(b_m1), W_m2, row(b_m2),
                Wu1h, Wu1a, row(b_u1), W_u2, row(b_u2),
                W_s1, row(b_s1), W_s2, row(b_s2),
                Wl1s, Wl1d, Wl1e, row(b_l1), W_l2, row(b_l2)]

    def bspec(x):
        if x.ndim == 3:
            return pl.BlockSpec((1,) + x.shape[1:], lambda p: (p, 0, 0))
        return pl.BlockSpec(x.shape, lambda p: (0,) * x.ndim)

    packed_rows = _BSL * _MAX_N

    cost_p, rmax_p, nd3, ed3 = pl.pallas_call(
        _stage_a,
        grid=(nblocks,),
        in_specs=[bspec(x) for x in a_inputs],
        out_specs=[
            pl.BlockSpec((packed_rows, 128), lambda p: (p, 0)),
            pl.BlockSpec((packed_rows, 128), lambda p: (p, 0)),
            pl.BlockSpec((_PPB, _MAX_N, _MAX_N), lambda p: (p, 0, 0)),
            pl.BlockSpec((_PPB, _MAX_E, _MAX_E), lambda p: (p, 0, 0)),
        ],
        out_shape=[
            jax.ShapeDtypeStruct((_SLABS * _MAX_N, 128), f32),
            jax.ShapeDtypeStruct((_SLABS * _MAX_N, 128), f32),
            jax.ShapeDtypeStruct((_PAIRS, _MAX_N, _MAX_N), f32),
            jax.ShapeDtypeStruct((_PAIRS, _MAX_E, _MAX_E), f32),
        ],
    )(*a_inputs)

    Pp = pl.pallas_call(
        _stage_b,
        grid=(1,),
        in_specs=[pl.BlockSpec((_SLABS * _MAX_N, 128), lambda p: (0, 0)),
                  pl.BlockSpec((_SLABS * _MAX_N, 128), lambda p: (0, 0))],
        out_specs=pl.BlockSpec((_SLABS * _MAX_N, 128), lambda p: (0, 0)),
        out_shape=jax.ShapeDtypeStruct((_SLABS * _MAX_N, 128), f32),
    )(cost_p, rmax_p)

    c_inputs = [Pp, nd3, ed3, qf, qt, cf, ct]
    c_specs = [
        pl.BlockSpec((packed_rows, 128), lambda p: (p, 0)),
        pl.BlockSpec((_PPB, _MAX_N, _MAX_N), lambda p: (p, 0, 0)),
        pl.BlockSpec((_PPB, _MAX_E, _MAX_E), lambda p: (p, 0, 0)),
        pl.BlockSpec((1, _PPB, _MAX_E), lambda p: (p, 0, 0)),
        pl.BlockSpec((1, _PPB, _MAX_E), lambda p: (p, 0, 0)),
        pl.BlockSpec((1, _PPB, _MAX_E), lambda p: (p, 0, 0)),
        pl.BlockSpec((1, _PPB, _MAX_E), lambda p: (p, 0, 0)),
    ]
    out3 = pl.pallas_call(
        _stage_c,
        grid=(nblocks,),
        in_specs=c_specs,
        out_specs=pl.BlockSpec((_PPB, 8, 128), lambda p: (p, 0, 0)),
        out_shape=jax.ShapeDtypeStruct((_PAIRS, 8, 128), f32),
    )(*c_inputs)
    return out3[:, 0, 0]
